# Initial kernel scaffold; baseline (speedup 1.0000x reference)
import jax
import jax.numpy as jnp
from jax.experimental import pallas as pl


def kernel(x):
    raise NotImplementedError("write your pallas kernel here")


# speedup vs baseline: 10.6446x; 10.6446x over previous
import jax
import jax.numpy as jnp
from jax.experimental import pallas as pl

_TEMP = 0.1
_TOPK = 32
_N = 4096
_D = 256
_BLK = 256


def _raw_sim(xb_ref, xf_ref):
    return jax.lax.dot_general(
        xb_ref[...], xf_ref[...],
        dimension_numbers=(((1,), (1,)), ((), ())),
        preferred_element_type=jnp.float32,
    )


def _thr_kernel(xb_ref, xf_ref, thr_ref):
    raw = _raw_sim(xb_ref, xf_ref)

    def knock_out(_, s):
        m = jnp.max(s, axis=1, keepdims=True)
        return jnp.where(s >= m, -jnp.inf, s)

    s = jax.lax.fori_loop(0, _TOPK - 1, knock_out, raw)
    thr_ref[...] = jnp.max(s, axis=1, keepdims=True)


def _adj_kernel(xb_ref, xf_ref, tcol_ref, trow_ref, out_ref):
    raw = _raw_sim(xb_ref, xf_ref)
    in_row = (raw >= tcol_ref[...]).astype(jnp.float32)
    in_col = (raw >= trow_ref[...]).astype(jnp.float32)
    out_ref[...] = (raw / jnp.float32(_TEMP)) * ((in_row + in_col) * 0.5)


def kernel(x):
    nblk = _N // _BLK
    thr = pl.pallas_call(
        _thr_kernel,
        grid=(nblk,),
        in_specs=[
            pl.BlockSpec((_BLK, _D), lambda i: (i, 0)),
            pl.BlockSpec((_N, _D), lambda i: (0, 0)),
        ],
        out_specs=pl.BlockSpec((_BLK, 1), lambda i: (i, 0)),
        out_shape=jax.ShapeDtypeStruct((_N, 1), jnp.float32),
    )(x, x)

    trow = thr.reshape(1, _N)

    adj = pl.pallas_call(
        _adj_kernel,
        grid=(nblk,),
        in_specs=[
            pl.BlockSpec((_BLK, _D), lambda i: (i, 0)),
            pl.BlockSpec((_N, _D), lambda i: (0, 0)),
            pl.BlockSpec((_BLK, 1), lambda i: (i, 0)),
            pl.BlockSpec((1, _N), lambda i: (0, 0)),
        ],
        out_specs=pl.BlockSpec((_BLK, _N), lambda i: (i, 0)),
        out_shape=jax.ShapeDtypeStruct((_N, _N), jnp.float32),
    )(x, x, thr, trow)
    return adj
